# baseline (speedup 1.0000x reference)
import functools

import jax
import jax.numpy as jnp
from jax import lax
from jax.experimental import pallas as pl
from jax.experimental.pallas import tpu as pltpu
from jax.experimental.pallas import tpu_sc as plsc

_LANES = 16


def _build_mask_sc(idx_padded, T):
    n_pad = idx_padded.shape[0]
    mesh = plsc.VectorSubcoreMesh(core_axis_name="c", subcore_axis_name="s")

    @functools.partial(
        pl.kernel,
        mesh=mesh,
        out_type=jax.ShapeDtypeStruct((T,), jnp.float32),
        scratch_types=[
            pltpu.VMEM((T,), jnp.float32),
            pltpu.VMEM((n_pad,), jnp.int32),
        ],
        compiler_params=pltpu.CompilerParams(needs_layout_passes=False),
    )
    def build(idx_hbm, mask_hbm, mask_v, idx_v):
        wid = lax.axis_index("s") * 2 + lax.axis_index("c")

        @pl.when(wid == 0)
        def _():
            ones = jnp.ones((_LANES,), jnp.float32)
            for i in range(T // _LANES):
                mask_v[pl.ds(i * _LANES, _LANES)] = ones
            pltpu.sync_copy(idx_hbm, idx_v)
            zeros = jnp.zeros((_LANES,), jnp.float32)
            for i in range(n_pad // _LANES):
                iv = idx_v[pl.ds(i * _LANES, _LANES)]
                plsc.store_scatter(mask_v, [iv], zeros)
            pltpu.sync_copy(mask_v, mask_hbm)

    return build(idx_padded)


_ROW_BLK = 256
_NBUF = 4


def _apply_mask_tc(x2d, mask2d):
    R, T = x2d.shape
    n = R // _ROW_BLK
    hbm = pl.BlockSpec(memory_space=pltpu.MemorySpace.HBM)

    def body(x_hbm, m_vmem, o_hbm, *scratch):
        ibufs = scratch[:_NBUF]
        obufs = scratch[_NBUF:2 * _NBUF]
        in_sems, out_sems = scratch[2 * _NBUF], scratch[2 * _NBUF + 1]

        def in_cp(i):
            k = i % _NBUF
            return pltpu.make_async_copy(
                x_hbm.at[pl.ds(i * _ROW_BLK, _ROW_BLK), :],
                ibufs[k], in_sems.at[k])

        def out_cp(i):
            k = i % _NBUF
            return pltpu.make_async_copy(
                obufs[k], o_hbm.at[pl.ds(i * _ROW_BLK, _ROW_BLK), :],
                out_sems.at[k])

        m = m_vmem[...]
        for i in range(_NBUF):
            in_cp(i).start()
        for i in range(n):
            k = i % _NBUF
            in_cp(i).wait()
            if i >= _NBUF:
                out_cp(i - _NBUF).wait()
            obufs[k][...] = ibufs[k][...] * m
            out_cp(i).start()
            if i + _NBUF < n:
                in_cp(i + _NBUF).start()
        for i in range(n - _NBUF, n):
            out_cp(i).wait()

    return pl.pallas_call(
        body,
        in_specs=[hbm, pl.BlockSpec(memory_space=pltpu.MemorySpace.VMEM)],
        out_specs=hbm,
        out_shape=jax.ShapeDtypeStruct((R, T), jnp.float32),
        scratch_shapes=(
            [pltpu.VMEM((_ROW_BLK, T), jnp.float32) for _ in range(2 * _NBUF)]
            + [pltpu.SemaphoreType.DMA((_NBUF,)),
               pltpu.SemaphoreType.DMA((_NBUF,))]
        ),
    )(x2d, mask2d)


def _probe_hbm_hbm(x2d):
    R, T = x2d.shape
    nc = 12
    blk = R // nc
    hbm = pl.BlockSpec(memory_space=pltpu.MemorySpace.HBM)

    def body(x_ref, o_ref, sems):
        cps = [
            pltpu.make_async_copy(
                x_ref.at[pl.ds(i * blk, blk), :],
                o_ref.at[pl.ds(i * blk, blk), :],
                sems.at[i],
            )
            for i in range(nc)
        ]
        for cp in cps:
            cp.start()
        for cp in cps:
            cp.wait()

    return pl.pallas_call(
        body,
        in_specs=[hbm],
        out_specs=hbm,
        out_shape=jax.ShapeDtypeStruct((R, T), jnp.float32),
        scratch_shapes=[pltpu.SemaphoreType.DMA((nc,))],
    )(x2d)


def kernel(x, mask_indices):
    B, C, T = x.shape
    n = mask_indices.shape[0]
    pad = (-n) % _LANES
    if pad:
        idx_padded = jnp.concatenate(
            [mask_indices, jnp.broadcast_to(mask_indices[:1], (pad,))]
        )
    else:
        idx_padded = mask_indices
    out2d = _probe_hbm_hbm(x.reshape(B * C, T))
    return out2d.reshape(B, C, T)


# speedup vs baseline: 42.1426x; 42.1426x over previous
import functools

import jax
import jax.numpy as jnp
from jax import lax
from jax.experimental import pallas as pl
from jax.experimental.pallas import tpu as pltpu
from jax.experimental.pallas import tpu_sc as plsc

_LANES = 16


def _build_mask_sc(idx_padded, T):
    n_pad = idx_padded.shape[0]
    mesh = plsc.VectorSubcoreMesh(core_axis_name="c", subcore_axis_name="s")

    @functools.partial(
        pl.kernel,
        mesh=mesh,
        out_type=jax.ShapeDtypeStruct((T,), jnp.float32),
        scratch_types=[
            pltpu.VMEM((T,), jnp.float32),
            pltpu.VMEM((n_pad,), jnp.int32),
        ],
        compiler_params=pltpu.CompilerParams(needs_layout_passes=False),
    )
    def build(idx_hbm, mask_hbm, mask_v, idx_v):
        wid = lax.axis_index("s") * 2 + lax.axis_index("c")

        @pl.when(wid == 0)
        def _():
            ones = jnp.ones((_LANES,), jnp.float32)
            for i in range(T // _LANES):
                mask_v[pl.ds(i * _LANES, _LANES)] = ones
            pltpu.sync_copy(idx_hbm, idx_v)
            zeros = jnp.zeros((_LANES,), jnp.float32)
            for i in range(n_pad // _LANES):
                iv = idx_v[pl.ds(i * _LANES, _LANES)]
                plsc.store_scatter(mask_v, [iv], zeros)
            pltpu.sync_copy(mask_v, mask_hbm)

    return build(idx_padded)


_ROW_BLK = 768


def _apply_mask_tc(x2d, mask2d):
    R, T = x2d.shape

    def body(x_ref, m_ref, o_ref):
        o_ref[...] = x_ref[...] * m_ref[...]

    return pl.pallas_call(
        body,
        grid=(R // _ROW_BLK,),
        in_specs=[
            pl.BlockSpec((_ROW_BLK, T), lambda i: (i, 0)),
            pl.BlockSpec((1, T), lambda i: (0, 0)),
        ],
        out_specs=pl.BlockSpec((_ROW_BLK, T), lambda i: (i, 0)),
        out_shape=jax.ShapeDtypeStruct((R, T), jnp.float32),
        compiler_params=pltpu.CompilerParams(
            dimension_semantics=("arbitrary",),
        ),
    )(x2d, mask2d)


def kernel(x, mask_indices):
    B, C, T = x.shape
    n = mask_indices.shape[0]
    pad = (-n) % _LANES
    if pad:
        idx_padded = jnp.concatenate(
            [mask_indices, jnp.broadcast_to(mask_indices[:1], (pad,))]
        )
    else:
        idx_padded = mask_indices
    mask = _build_mask_sc(idx_padded, T)
    out2d = _apply_mask_tc(x.reshape(B * C, T), mask.reshape(1, T))
    return out2d.reshape(B, C, T)
